# baseline (speedup 1.0000x reference)
import functools

import jax
import jax.numpy as jnp
from jax.experimental import pallas as pl
from jax.experimental.pallas import tpu as pltpu

_N_BINS = 15
_ROWS_PER_BLOCK = 1000


def _ece_kernel(nblocks, n_total, x_ref, lab_ref, bnd_ref, out_ref, acc_ref):
    i = pl.program_id(0)

    @pl.when(i == 0)
    def _init():
        acc_ref[...] = jnp.zeros_like(acc_ref)

    x = x_ref[...]
    R, C = x.shape
    m = jnp.max(x, axis=1, keepdims=True)
    s = jnp.sum(jnp.exp(x - m), axis=1, keepdims=True)
    conf = 1.0 / s
    col = jax.lax.broadcasted_iota(jnp.int32, (R, C), 1)
    pred = jnp.min(jnp.where(x == m, col, C), axis=1, keepdims=True)
    lbl = lab_ref[0]
    acc = (pred == lbl).astype(jnp.float32)

    cmp = (conf > bnd_ref[0:1, 1:_N_BINS]).astype(jnp.float32)
    binidx = jnp.sum(cmp, axis=1, keepdims=True).astype(jnp.int32)
    jrow = jax.lax.broadcasted_iota(jnp.int32, (R, _N_BINS), 1)
    onehot = (binidx == jrow).astype(jnp.float32)
    acc_ref[0:1, 0:_N_BINS] += jnp.sum(onehot, axis=0, keepdims=True)
    acc_ref[1:2, 0:_N_BINS] += jnp.sum(onehot * conf, axis=0, keepdims=True)
    acc_ref[2:3, 0:_N_BINS] += jnp.sum(onehot * acc, axis=0, keepdims=True)

    @pl.when(i == nblocks - 1)
    def _fin():
        cnt = acc_ref[0:1, 0:_N_BINS]
        sc = acc_ref[1:2, 0:_N_BINS]
        sa = acc_ref[2:3, 0:_N_BINS]
        denom = jnp.maximum(cnt, 1.0)
        term = jnp.where(cnt > 0.0,
                         jnp.abs(sc / denom - sa / denom) * (cnt / n_total),
                         0.0)
        out_ref[...] = jnp.sum(term).reshape(1, 1)


def kernel(logits, labels):
    n, c = logits.shape
    r = _ROWS_PER_BLOCK
    nblocks = n // r
    labels3 = labels.astype(jnp.int32).reshape(nblocks, r, 1)
    bnd = jnp.linspace(0.0, 1.0, _N_BINS + 1).astype(jnp.float32).reshape(1, _N_BINS + 1)

    out = pl.pallas_call(
        functools.partial(_ece_kernel, nblocks, float(n)),
        grid=(nblocks,),
        in_specs=[
            pl.BlockSpec((r, c), lambda i: (i, 0)),
            pl.BlockSpec((1, r, 1), lambda i: (i, 0, 0)),
            pl.BlockSpec((1, _N_BINS + 1), lambda i: (0, 0)),
        ],
        out_specs=pl.BlockSpec((1, 1), lambda i: (0, 0)),
        out_shape=jax.ShapeDtypeStruct((1, 1), jnp.float32),
        scratch_shapes=[pltpu.VMEM((8, 128), jnp.float32)],
    )(logits, labels3, bnd)
    return out.reshape(1)


# speedup vs baseline: 1.0494x; 1.0494x over previous
import functools

import jax
import jax.numpy as jnp
from jax.experimental import pallas as pl
from jax.experimental.pallas import tpu as pltpu

_N_BINS = 15
_ROWS_PER_BLOCK = 2000


def _ece_kernel(nblocks, n_total, x_ref, lab_ref, bnd_ref, out_ref, acc_ref):
    i = pl.program_id(0)

    @pl.when(i == 0)
    def _init():
        acc_ref[...] = jnp.zeros_like(acc_ref)

    x = x_ref[...]
    R, C = x.shape
    m = jnp.max(x, axis=1, keepdims=True)
    s = jnp.sum(jnp.exp(x - m), axis=1, keepdims=True)
    conf = 1.0 / s
    col = jax.lax.broadcasted_iota(jnp.int32, (R, C), 1)
    pred = jnp.min(jnp.where(x == m, col, C), axis=1, keepdims=True)
    lbl = lab_ref[0]
    acc = (pred == lbl).astype(jnp.float32)

    cmp = (conf > bnd_ref[0:1, 1:_N_BINS]).astype(jnp.float32)
    binidx = jnp.sum(cmp, axis=1, keepdims=True).astype(jnp.int32)
    jrow = jax.lax.broadcasted_iota(jnp.int32, (R, _N_BINS), 1)
    onehot = (binidx == jrow).astype(jnp.float32)
    acc_ref[0:1, 0:_N_BINS] += jnp.sum(onehot, axis=0, keepdims=True)
    acc_ref[1:2, 0:_N_BINS] += jnp.sum(onehot * conf, axis=0, keepdims=True)
    acc_ref[2:3, 0:_N_BINS] += jnp.sum(onehot * acc, axis=0, keepdims=True)

    @pl.when(i == nblocks - 1)
    def _fin():
        cnt = acc_ref[0:1, 0:_N_BINS]
        sc = acc_ref[1:2, 0:_N_BINS]
        sa = acc_ref[2:3, 0:_N_BINS]
        denom = jnp.maximum(cnt, 1.0)
        term = jnp.where(cnt > 0.0,
                         jnp.abs(sc / denom - sa / denom) * (cnt / n_total),
                         0.0)
        out_ref[...] = jnp.sum(term).reshape(1, 1)


def kernel(logits, labels):
    n, c = logits.shape
    r = _ROWS_PER_BLOCK
    nblocks = n // r
    labels3 = labels.astype(jnp.int32).reshape(nblocks, r, 1)
    bnd = jnp.linspace(0.0, 1.0, _N_BINS + 1).astype(jnp.float32).reshape(1, _N_BINS + 1)

    out = pl.pallas_call(
        functools.partial(_ece_kernel, nblocks, float(n)),
        grid=(nblocks,),
        in_specs=[
            pl.BlockSpec((r, c), lambda i: (i, 0)),
            pl.BlockSpec((1, r, 1), lambda i: (i, 0, 0)),
            pl.BlockSpec((1, _N_BINS + 1), lambda i: (0, 0)),
        ],
        out_specs=pl.BlockSpec((1, 1), lambda i: (0, 0)),
        out_shape=jax.ShapeDtypeStruct((1, 1), jnp.float32),
        scratch_shapes=[pltpu.VMEM((8, 128), jnp.float32)],
    )(logits, labels3, bnd)
    return out.reshape(1)
